# baseline (speedup 1.0000x reference)
import functools

import jax
import jax.numpy as jnp
from jax import lax
from jax.experimental import pallas as pl
from jax.experimental.pallas import tpu as pltpu
from jax.experimental.pallas import tpu_sc as plsc

N = 10000
NP = 10240
E = 320000
NC, NS = 2, 16
NW = NC * NS
CH = 80
EPW = E // NW
NCHUNK = EPW // CH
RPT = NP // NS
ZR = 80
BR = 1000


def _make_sc_agg(D):
  mesh = plsc.VectorSubcoreMesh(core_axis_name="c", subcore_axis_name="s",
                                num_cores=NC, num_subcores=NS)

  @functools.partial(
      pl.kernel, mesh=mesh,
      out_type=jax.ShapeDtypeStruct((NC, NP, D), jnp.float32),
      scratch_types=(
          pltpu.VMEM_SHARED((NP, D), jnp.float32),
          pltpu.VMEM((CH,), jnp.int32),
          pltpu.VMEM((CH,), jnp.int32),
          pltpu.VMEM((CH,), jnp.int32),
          pltpu.VMEM((CH,), jnp.int32),
          pltpu.VMEM((CH, D), jnp.float32),
          pltpu.VMEM((CH, D), jnp.float32),
          pltpu.SemaphoreType.DMA,
          pltpu.SemaphoreType.DMA,
          pltpu.SemaphoreType.DMA,
          pltpu.SemaphoreType.DMA,
      ))
  def sc_agg(h_hbm, src_hbm, dst_hbm, zd_hbm, out_hbm,
             acc, src0, src1, dst0, dst1, rows0, rows1,
             semg0, semg1, semi0, semi1):
    cid = lax.axis_index("c")
    sid = lax.axis_index("s")
    wid = sid * NC + cid
    tile_row0 = sid * RPT
    base = wid * EPW
    srcs = (src0, src1)
    dsts = (dst0, dst1)
    rows = (rows0, rows1)
    semg = (semg0, semg1)
    semi = (semi0, semi1)

    for j in range(RPT // ZR):
      pltpu.sync_copy(zd_hbm, acc.at[pl.ds(tile_row0 + j * ZR, ZR)])
    plsc.subcore_barrier()

    def start_idx(c, b):
      off = base + jnp.minimum(c, NCHUNK - 1) * CH
      pltpu.async_copy(src_hbm.at[pl.ds(off, CH)], srcs[b], semi[b])
      pltpu.async_copy(dst_hbm.at[pl.ds(off, CH)], dsts[b], semi[b])

    def wait_idx(b):
      pltpu.make_async_copy(src_hbm.at[pl.ds(0, CH)], srcs[b], semi[b]).wait()
      pltpu.make_async_copy(dst_hbm.at[pl.ds(0, CH)], dsts[b], semi[b]).wait()

    def start_gather(b):
      pltpu.async_copy(h_hbm.at[srcs[b]], rows[b], semg[b])

    def wait_gather(b):
      pltpu.make_async_copy(h_hbm.at[srcs[b]], rows[b], semg[b]).wait()

    def half(b):
      wait_idx(1 - b)
      start_gather(1 - b)
      wait_gather(b)
      pltpu.sync_copy(rows[b], acc.at[dsts[b]], add=True)

    pltpu.sync_copy(src_hbm.at[pl.ds(base, CH)], src0)
    pltpu.sync_copy(dst_hbm.at[pl.ds(base, CH)], dst0)
    start_gather(0)
    start_idx(1, 1)

    def pair(k, carry):
      start_idx_c2 = 2 * k + 2
      half(0)
      start_idx(start_idx_c2, 0)
      half(1)
      start_idx(start_idx_c2 + 1, 1)
      return carry

    lax.fori_loop(0, (NCHUNK - 1) // 2, pair, 0)
    wait_idx(1)
    wait_gather(0)
    pltpu.sync_copy(rows[0], acc.at[dsts[0]], add=True)
    plsc.subcore_barrier()

    pltpu.sync_copy(acc.at[pl.ds(tile_row0, RPT)],
                    out_hbm.at[cid, pl.ds(tile_row0, RPT)])

  return sc_agg


def _make_sc_cnt():
  mesh = plsc.VectorSubcoreMesh(core_axis_name="c", subcore_axis_name="s",
                                num_cores=NC, num_subcores=NS)

  @functools.partial(
      pl.kernel, mesh=mesh,
      out_type=jax.ShapeDtypeStruct((NC, NP, 128), jnp.float32),
      scratch_types=(
          pltpu.VMEM_SHARED((NP, 128), jnp.float32),
          pltpu.VMEM((CH,), jnp.int32),
          pltpu.VMEM((CH,), jnp.int32),
          pltpu.VMEM((CH, 128), jnp.float32),
          pltpu.SemaphoreType.DMA,
          pltpu.SemaphoreType.DMA,
      ))
  def sc_cnt(dst_hbm, zd_hbm, ones_hbm, out_hbm,
             acc, dst0, dst1, ones_v, semi0, semi1):
    cid = lax.axis_index("c")
    sid = lax.axis_index("s")
    wid = sid * NC + cid
    tile_row0 = sid * RPT
    base = wid * EPW
    dsts = (dst0, dst1)
    semi = (semi0, semi1)

    for j in range(RPT // ZR):
      pltpu.sync_copy(zd_hbm, acc.at[pl.ds(tile_row0 + j * ZR, ZR)])
    pltpu.sync_copy(ones_hbm, ones_v)
    plsc.subcore_barrier()

    def start_idx(c, b):
      off = base + jnp.minimum(c, NCHUNK - 1) * CH
      pltpu.async_copy(dst_hbm.at[pl.ds(off, CH)], dsts[b], semi[b])

    def wait_idx(b):
      pltpu.make_async_copy(dst_hbm.at[pl.ds(0, CH)], dsts[b], semi[b]).wait()

    def half(c, b):
      wait_idx(b)
      pltpu.sync_copy(ones_v, acc.at[dsts[b]], add=True)
      start_idx(c + 2, b)

    start_idx(0, 0)
    start_idx(1, 1)

    def pair(k, carry):
      half(2 * k, 0)
      half(2 * k + 1, 1)
      return carry

    lax.fori_loop(0, (NCHUNK - 1) // 2, pair, 0)
    wait_idx(0)
    pltpu.sync_copy(ones_v, acc.at[dsts[0]], add=True)
    wait_idx(1)
    plsc.subcore_barrier()

    pltpu.sync_copy(acc.at[pl.ds(tile_row0, RPT)],
                    out_hbm.at[cid, pl.ds(tile_row0, RPT)])

  return sc_cnt


_SC_CACHE = {}


def _sc_agg(D):
  if D not in _SC_CACHE:
    _SC_CACHE[D] = _make_sc_agg(D)
  return _SC_CACHE[D]


def _sc_cnt():
  if "cnt" not in _SC_CACHE:
    _SC_CACHE["cnt"] = _make_sc_cnt()
  return _SC_CACHE["cnt"]


def _leaky(v):
  return jnp.where(v > 0, v, 0.15 * v)


def _inv_cnt(cntp):
  c = cntp[0, :, 0] + cntp[1, :, 0]
  return 1.0 / jnp.maximum(c, 1.0)


def _tc1_body(accp, cntp, x, wl, bl, wr, out):
  mean = (accp[0] + accp[1]) * _inv_cnt(cntp)[:, None]
  h = (jnp.dot(mean, wl[...], preferred_element_type=jnp.float32) + bl[0]
       + jnp.dot(x[...], wr[...], preferred_element_type=jnp.float32))
  out[...] = _leaky(h)


def _tc2_body(accp, cntp, h1, wl2, bl2, wr2, wl3, wr3, bl3, t3, r3):
  mean = (accp[0] + accp[1]) * _inv_cnt(cntp)[:, None]
  h2 = _leaky(jnp.dot(mean, wl2[...], preferred_element_type=jnp.float32) + bl2[0]
              + jnp.dot(h1[...], wr2[...], preferred_element_type=jnp.float32))
  t = jnp.dot(h2, wl3[...], preferred_element_type=jnp.float32)
  t3[...] = jnp.concatenate([t, jnp.zeros_like(t)], axis=1)
  r3[...] = jnp.dot(h2, wr3[...], preferred_element_type=jnp.float32) + bl3[0]


def _tc3_body(accp, cntp, r3, wp, bp, w1, b1, w2, b2, out):
  mean3 = (accp[0, :, :64] + accp[1, :, :64]) * _inv_cnt(cntp)[:, None]
  h3 = _leaky(mean3 + r3[...])
  h4 = jnp.dot(h3, wp[...], preferred_element_type=jnp.float32) + bp[0]
  h5 = _leaky(jnp.dot(h4, w1[...], preferred_element_type=jnp.float32) + b1[0])
  out[...] = jnp.dot(h5, w2[...], preferred_element_type=jnp.float32) + b2[0]


def _row_block3(d):
  return pl.BlockSpec((NC, BR, d), lambda i: (0, i, 0))


def _row_block(d):
  return pl.BlockSpec((BR, d), lambda i: (i, 0))


def _full(*dims):
  return pl.BlockSpec(dims, lambda i: tuple(0 for _ in dims))


def _tc1(accp, cntp, x, wl, bl, wr):
  return pl.pallas_call(
      _tc1_body,
      grid=(N // BR,),
      in_specs=[_row_block3(128), _row_block3(128), _row_block(128),
                _full(128, 128), _full(1, 128), _full(128, 128)],
      out_specs=_row_block(128),
      out_shape=jax.ShapeDtypeStruct((N, 128), jnp.float32),
  )(accp, cntp, x, wl, bl, wr)


def _tc2(accp, cntp, h1, wl2, bl2, wr2, wl3, wr3, bl3):
  return pl.pallas_call(
      _tc2_body,
      grid=(N // BR,),
      in_specs=[_row_block3(128), _row_block3(128), _row_block(128),
                _full(128, 256), _full(1, 256), _full(128, 256),
                _full(256, 64), _full(256, 64), _full(1, 64)],
      out_specs=[_row_block(128), _row_block(64)],
      out_shape=[jax.ShapeDtypeStruct((N, 128), jnp.float32),
                 jax.ShapeDtypeStruct((N, 64), jnp.float32)],
  )(accp, cntp, h1, wl2, bl2, wr2, wl3, wr3, bl3)


def _tc3(accp, cntp, r3, wp, bp, w1, b1, w2, b2):
  return pl.pallas_call(
      _tc3_body,
      grid=(N // BR,),
      in_specs=[_row_block3(128), _row_block3(128), _row_block(64),
                _full(64, 32), _full(1, 32), _full(32, 32), _full(1, 32),
                _full(32, 2), _full(1, 2)],
      out_specs=_row_block(2),
      out_shape=jax.ShapeDtypeStruct((N, 2), jnp.float32),
  )(accp, cntp, r3, wp, bp, w1, b1, w2, b2)


def kernel(x, edge_index, edge_attr, W_edge, b_edge, Wl1, bl1, Wr1, Wl2, bl2,
           Wr2, Wl3, bl3, Wr3, W_pre, b_pre, W_fc1, b_fc1, W_fc2, b_fc2):
  src = edge_index[0]
  dst = edge_index[1]

  zd128 = jnp.zeros((ZR, 128), jnp.float32)
  ones = jnp.ones((CH, 128), jnp.float32)

  cntp = _sc_cnt()(dst, zd128, ones)
  acc1 = _sc_agg(128)(x, src, dst, zd128)
  h1 = _tc1(acc1, cntp, x, Wl1, bl1.reshape(1, -1), Wr1)

  acc2 = _sc_agg(128)(h1, src, dst, zd128)
  t3, r3 = _tc2(acc2, cntp, h1, Wl2, bl2.reshape(1, -1), Wr2,
                Wl3, Wr3, bl3.reshape(1, -1))

  acc3 = _sc_agg(128)(t3, src, dst, zd128)
  out = _tc3(acc3, cntp, r3, W_pre, b_pre.reshape(1, -1),
             W_fc1, b_fc1.reshape(1, -1), W_fc2, b_fc2.reshape(1, -1))
  return out


# speedup vs baseline: 1.0844x; 1.0844x over previous
import functools

import jax
import jax.numpy as jnp
from jax import lax
from jax.experimental import pallas as pl
from jax.experimental.pallas import tpu as pltpu
from jax.experimental.pallas import tpu_sc as plsc

N = 10000
NP = 10240
E = 320000
NC, NS = 2, 16
NW = NC * NS
CH = 80
EPW = E // NW
NCHUNK = EPW // CH
RPT = NP // NS
ZR = 80
BR = 1000


def _make_sc_agg(D):
  mesh = plsc.VectorSubcoreMesh(core_axis_name="c", subcore_axis_name="s",
                                num_cores=NC, num_subcores=NS)

  @functools.partial(
      pl.kernel, mesh=mesh,
      out_type=jax.ShapeDtypeStruct((NC, NP, D), jnp.float32),
      scratch_types=(
          pltpu.VMEM_SHARED((NP, D), jnp.float32),
          pltpu.VMEM((CH,), jnp.int32),
          pltpu.VMEM((CH,), jnp.int32),
          pltpu.VMEM((CH,), jnp.int32),
          pltpu.VMEM((CH,), jnp.int32),
          pltpu.VMEM((CH, D), jnp.float32),
          pltpu.VMEM((CH, D), jnp.float32),
          pltpu.SemaphoreType.DMA,
          pltpu.SemaphoreType.DMA,
          pltpu.SemaphoreType.DMA,
          pltpu.SemaphoreType.DMA,
      ))
  def sc_agg(h_hbm, src_hbm, dst_hbm, zd_hbm, out_hbm,
             acc, src0, src1, dst0, dst1, rows0, rows1,
             semg0, semg1, semi0, semi1):
    cid = lax.axis_index("c")
    sid = lax.axis_index("s")
    wid = sid * NC + cid
    tile_row0 = sid * RPT
    base = wid * EPW
    srcs = (src0, src1)
    dsts = (dst0, dst1)
    rows = (rows0, rows1)
    semg = (semg0, semg1)
    semi = (semi0, semi1)

    pltpu.sync_copy(zd_hbm, rows0)
    for j in range(RPT // ZR):
      pltpu.sync_copy(rows0, acc.at[pl.ds(tile_row0 + j * ZR, ZR)])
    plsc.subcore_barrier()

    def start_idx(c, b):
      off = base + jnp.minimum(c, NCHUNK - 1) * CH
      pltpu.async_copy(src_hbm.at[pl.ds(off, CH)], srcs[b], semi[b])
      pltpu.async_copy(dst_hbm.at[pl.ds(off, CH)], dsts[b], semi[b])

    def wait_idx(b):
      pltpu.make_async_copy(src_hbm.at[pl.ds(0, CH)], srcs[b], semi[b]).wait()
      pltpu.make_async_copy(dst_hbm.at[pl.ds(0, CH)], dsts[b], semi[b]).wait()

    def start_gather(b):
      pltpu.async_copy(h_hbm.at[srcs[b]], rows[b], semg[b])

    def wait_gather(b):
      pltpu.make_async_copy(h_hbm.at[srcs[b]], rows[b], semg[b]).wait()

    def half(b):
      wait_idx(1 - b)
      start_gather(1 - b)
      wait_gather(b)
      pltpu.sync_copy(rows[b], acc.at[dsts[b]], add=True)

    pltpu.sync_copy(src_hbm.at[pl.ds(base, CH)], src0)
    pltpu.sync_copy(dst_hbm.at[pl.ds(base, CH)], dst0)
    start_gather(0)
    start_idx(1, 1)

    def pair(k, carry):
      start_idx_c2 = 2 * k + 2
      half(0)
      start_idx(start_idx_c2, 0)
      half(1)
      start_idx(start_idx_c2 + 1, 1)
      return carry

    lax.fori_loop(0, (NCHUNK - 1) // 2, pair, 0)
    wait_idx(1)
    wait_gather(0)
    pltpu.sync_copy(rows[0], acc.at[dsts[0]], add=True)
    plsc.subcore_barrier()

    for j in range(RPT // ZR):
      r0 = tile_row0 + j * ZR
      pltpu.sync_copy(acc.at[pl.ds(r0, ZR)], rows0)
      pltpu.sync_copy(rows0, out_hbm.at[cid, pl.ds(r0, ZR)])

  return sc_agg


def _make_sc_cnt():
  mesh = plsc.VectorSubcoreMesh(core_axis_name="c", subcore_axis_name="s",
                                num_cores=NC, num_subcores=NS)

  @functools.partial(
      pl.kernel, mesh=mesh,
      out_type=jax.ShapeDtypeStruct((NC, NP, 128), jnp.float32),
      scratch_types=(
          pltpu.VMEM_SHARED((NP, 128), jnp.float32),
          pltpu.VMEM((CH,), jnp.int32),
          pltpu.VMEM((CH,), jnp.int32),
          pltpu.VMEM((CH, 128), jnp.float32),
          pltpu.SemaphoreType.DMA,
          pltpu.SemaphoreType.DMA,
      ))
  def sc_cnt(dst_hbm, zd_hbm, ones_hbm, out_hbm,
             acc, dst0, dst1, ones_v, semi0, semi1):
    cid = lax.axis_index("c")
    sid = lax.axis_index("s")
    wid = sid * NC + cid
    tile_row0 = sid * RPT
    base = wid * EPW
    dsts = (dst0, dst1)
    semi = (semi0, semi1)

    pltpu.sync_copy(zd_hbm, ones_v)
    for j in range(RPT // ZR):
      pltpu.sync_copy(ones_v, acc.at[pl.ds(tile_row0 + j * ZR, ZR)])
    pltpu.sync_copy(ones_hbm, ones_v)
    plsc.subcore_barrier()

    def start_idx(c, b):
      off = base + jnp.minimum(c, NCHUNK - 1) * CH
      pltpu.async_copy(dst_hbm.at[pl.ds(off, CH)], dsts[b], semi[b])

    def wait_idx(b):
      pltpu.make_async_copy(dst_hbm.at[pl.ds(0, CH)], dsts[b], semi[b]).wait()

    def half(c, b):
      wait_idx(b)
      pltpu.sync_copy(ones_v, acc.at[dsts[b]], add=True)
      start_idx(c + 2, b)

    start_idx(0, 0)
    start_idx(1, 1)

    def pair(k, carry):
      half(2 * k, 0)
      half(2 * k + 1, 1)
      return carry

    lax.fori_loop(0, (NCHUNK - 1) // 2, pair, 0)
    wait_idx(0)
    pltpu.sync_copy(ones_v, acc.at[dsts[0]], add=True)
    wait_idx(1)
    plsc.subcore_barrier()

    for j in range(RPT // ZR):
      r0 = tile_row0 + j * ZR
      pltpu.sync_copy(acc.at[pl.ds(r0, ZR)], ones_v)
      pltpu.sync_copy(ones_v, out_hbm.at[cid, pl.ds(r0, ZR)])

  return sc_cnt


_SC_CACHE = {}


def _sc_agg(D):
  if D not in _SC_CACHE:
    _SC_CACHE[D] = _make_sc_agg(D)
  return _SC_CACHE[D]


def _sc_cnt():
  if "cnt" not in _SC_CACHE:
    _SC_CACHE["cnt"] = _make_sc_cnt()
  return _SC_CACHE["cnt"]


def _leaky(v):
  return jnp.where(v > 0, v, 0.15 * v)


def _inv_cnt(cntp):
  c = cntp[0, :, 0] + cntp[1, :, 0]
  return 1.0 / jnp.maximum(c, 1.0)


def _tc1_body(accp, cntp, x, wl, bl, wr, out):
  mean = (accp[0] + accp[1]) * _inv_cnt(cntp)[:, None]
  h = (jnp.dot(mean, wl[...], preferred_element_type=jnp.float32) + bl[0]
       + jnp.dot(x[...], wr[...], preferred_element_type=jnp.float32))
  out[...] = _leaky(h)


def _tc2_body(accp, cntp, h1, wl2, bl2, wr2, wl3, wr3, bl3, t3, r3):
  mean = (accp[0] + accp[1]) * _inv_cnt(cntp)[:, None]
  h2 = _leaky(jnp.dot(mean, wl2[...], preferred_element_type=jnp.float32) + bl2[0]
              + jnp.dot(h1[...], wr2[...], preferred_element_type=jnp.float32))
  t = jnp.dot(h2, wl3[...], preferred_element_type=jnp.float32)
  t3[...] = jnp.concatenate([t, jnp.zeros_like(t)], axis=1)
  r3[...] = jnp.dot(h2, wr3[...], preferred_element_type=jnp.float32) + bl3[0]


def _tc3_body(accp, cntp, r3, wp, bp, w1, b1, w2, b2, out):
  mean3 = (accp[0, :, :64] + accp[1, :, :64]) * _inv_cnt(cntp)[:, None]
  h3 = _leaky(mean3 + r3[...])
  h4 = jnp.dot(h3, wp[...], preferred_element_type=jnp.float32) + bp[0]
  h5 = _leaky(jnp.dot(h4, w1[...], preferred_element_type=jnp.float32) + b1[0])
  out[...] = jnp.dot(h5, w2[...], preferred_element_type=jnp.float32) + b2[0]


def _row_block3(d):
  return pl.BlockSpec((NC, BR, d), lambda i: (0, i, 0))


def _row_block(d):
  return pl.BlockSpec((BR, d), lambda i: (i, 0))


def _full(*dims):
  return pl.BlockSpec(dims, lambda i: tuple(0 for _ in dims))


def _tc1(accp, cntp, x, wl, bl, wr):
  return pl.pallas_call(
      _tc1_body,
      grid=(N // BR,),
      in_specs=[_row_block3(128), _row_block3(128), _row_block(128),
                _full(128, 128), _full(1, 128), _full(128, 128)],
      out_specs=_row_block(128),
      out_shape=jax.ShapeDtypeStruct((N, 128), jnp.float32),
  )(accp, cntp, x, wl, bl, wr)


def _tc2(accp, cntp, h1, wl2, bl2, wr2, wl3, wr3, bl3):
  return pl.pallas_call(
      _tc2_body,
      grid=(N // BR,),
      in_specs=[_row_block3(128), _row_block3(128), _row_block(128),
                _full(128, 256), _full(1, 256), _full(128, 256),
                _full(256, 64), _full(256, 64), _full(1, 64)],
      out_specs=[_row_block(128), _row_block(64)],
      out_shape=[jax.ShapeDtypeStruct((N, 128), jnp.float32),
                 jax.ShapeDtypeStruct((N, 64), jnp.float32)],
  )(accp, cntp, h1, wl2, bl2, wr2, wl3, wr3, bl3)


def _tc3(accp, cntp, r3, wp, bp, w1, b1, w2, b2):
  return pl.pallas_call(
      _tc3_body,
      grid=(N // BR,),
      in_specs=[_row_block3(128), _row_block3(128), _row_block(64),
                _full(64, 32), _full(1, 32), _full(32, 32), _full(1, 32),
                _full(32, 2), _full(1, 2)],
      out_specs=_row_block(2),
      out_shape=jax.ShapeDtypeStruct((N, 2), jnp.float32),
  )(accp, cntp, r3, wp, bp, w1, b1, w2, b2)


def kernel(x, edge_index, edge_attr, W_edge, b_edge, Wl1, bl1, Wr1, Wl2, bl2,
           Wr2, Wl3, bl3, Wr3, W_pre, b_pre, W_fc1, b_fc1, W_fc2, b_fc2):
  src = edge_index[0]
  dst = edge_index[1]

  zd128 = jnp.zeros((ZR, 128), jnp.float32)
  ones = jnp.ones((CH, 128), jnp.float32)

  cntp = _sc_cnt()(dst, zd128, ones)
  acc1 = _sc_agg(128)(x, src, dst, zd128)
  h1 = _tc1(acc1, cntp, x, Wl1, bl1.reshape(1, -1), Wr1)

  acc2 = _sc_agg(128)(h1, src, dst, zd128)
  t3, r3 = _tc2(acc2, cntp, h1, Wl2, bl2.reshape(1, -1), Wr2,
                Wl3, Wr3, bl3.reshape(1, -1))

  acc3 = _sc_agg(128)(t3, src, dst, zd128)
  out = _tc3(acc3, cntp, r3, W_pre, b_pre.reshape(1, -1),
             W_fc1, b_fc1.reshape(1, -1), W_fc2, b_fc2.reshape(1, -1))
  return out
